# Initial kernel scaffold; baseline (speedup 1.0000x reference)
import jax
import jax.numpy as jnp
from jax.experimental import pallas as pl


def kernel(h, edge_index, W_node, b_node, w_att, w_scale, bias, ln_gamma, ln_beta):
    raise NotImplementedError("write your pallas kernel here")


# speedup vs baseline: 36.1738x; 36.1738x over previous
import functools

import jax
import jax.numpy as jnp
from jax import lax
from jax.experimental import pallas as pl
from jax.experimental.pallas import tpu as pltpu
from jax.experimental.pallas import tpu_sc as plsc

F32 = jnp.float32
I32 = jnp.int32
NEG_SLOPE = 0.2

_NC, _NS = 2, 16
_NW = _NC * _NS
_NP = 10240


@functools.lru_cache(maxsize=None)
def _mesh():
  return plsc.VectorSubcoreMesh(core_axis_name="c", subcore_axis_name="s",
                                num_cores=_NC, num_subcores=_NS)


def _splat(vec, j):
  idx = jnp.full((16,), j, I32)
  return lax.gather(
      vec, idx[:, None],
      lax.GatherDimensionNumbers(offset_dims=(), collapsed_slice_dims=(0,),
                                 start_index_map=(0,)),
      slice_sizes=(1,), mode=lax.GatherScatterMode.PROMISE_IN_BOUNDS)


def _tc_a_body(h_ref, w_ref, b_ref, a_ref, wv_ref, al_ref):
  hb = h_ref[...]
  wv = jnp.dot(hb, w_ref[...], preferred_element_type=F32) + b_ref[...]
  al = jnp.dot(wv, a_ref[...], preferred_element_type=F32)
  blk = hb.shape[0]
  wv_ref[...] = wv.reshape(blk, 3, 128).transpose(1, 0, 2)
  al_ref[...] = al


def _tc_a(h, w_node, b2, a_cat, n, blk):
  grid = n // blk
  return pl.pallas_call(
      _tc_a_body,
      grid=(grid,),
      in_specs=[
          pl.BlockSpec((blk, 128), lambda i: (i, 0)),
          pl.BlockSpec((128, 384), lambda i: (0, 0)),
          pl.BlockSpec((1, 384), lambda i: (0, 0)),
          pl.BlockSpec((384, 8), lambda i: (0, 0)),
      ],
      out_specs=[
          pl.BlockSpec((3, blk, 128), lambda i: (0, i, 0)),
          pl.BlockSpec((blk, 8), lambda i: (i, 0)),
      ],
      out_shape=[
          jax.ShapeDtypeStruct((3, n, 128), F32),
          jax.ShapeDtypeStruct((n, 8), F32),
      ],
  )(h, w_node, b2, a_cat)


def _tc_b2_body(dp_ref, out_ref):
  out_ref[...] = jnp.sum(dp_ref[...], axis=0, keepdims=True)


def _tc_b2(denom_p):
  blk = 1024
  grid = 3 * _NP // blk
  return pl.pallas_call(
      _tc_b2_body,
      grid=(grid,),
      in_specs=[pl.BlockSpec((_NC, blk), lambda i: (0, i))],
      out_specs=pl.BlockSpec((1, blk), lambda i: (0, i)),
      out_shape=jax.ShapeDtypeStruct((1, 3 * _NP), F32),
  )(denom_p)


def _tc_d_body(h0_ref, h1_ref, h2_ref, h_ref, ws_ref, b_ref, g_ref, be_ref,
               out_ref):
  ws = ws_ref[...]
  acc = jnp.dot(h0_ref[...], ws[0:128], preferred_element_type=F32)
  acc += jnp.dot(h1_ref[...], ws[128:256], preferred_element_type=F32)
  acc += jnp.dot(h2_ref[...], ws[256:384], preferred_element_type=F32)
  y = acc + b_ref[...] + h_ref[...]
  mu = jnp.mean(y, axis=-1, keepdims=True)
  d = y - mu
  var = jnp.mean(d * d, axis=-1, keepdims=True)
  ln = d * jax.lax.rsqrt(var + 1e-5) * g_ref[...] + be_ref[...]
  out_ref[...] = jnp.maximum(ln, 0.0)


def _tc_d(out1, out2, h, w_scale, b2, g2, be2, n, blk):
  grid = n // blk
  nb = n // blk
  return pl.pallas_call(
      _tc_d_body,
      grid=(grid,),
      in_specs=[
          pl.BlockSpec((blk, 128), lambda i: (i, 0)),
          pl.BlockSpec((blk, 128), lambda i: (i, 0)),
          pl.BlockSpec((blk, 128), lambda i: (i + nb, 0)),
          pl.BlockSpec((blk, 128), lambda i: (i, 0)),
          pl.BlockSpec((384, 128), lambda i: (0, 0)),
          pl.BlockSpec((1, 128), lambda i: (0, 0)),
          pl.BlockSpec((1, 128), lambda i: (0, 0)),
          pl.BlockSpec((1, 128), lambda i: (0, 0)),
      ],
      out_specs=pl.BlockSpec((blk, 128), lambda i: (i, 0)),
      out_shape=jax.ShapeDtypeStruct((n, 128), F32),
  )(out1, out2, out1, h, w_scale, b2, g2, be2)


def _sc_b_body(n, e, src_hbm, dst_hbm, alpha_hbm, ex_hbm, denom_hbm,
               alpha_v, src_v, dst_v, ex0_v, ex1_v, ex2_v,
               eb0_v, eb1_v, eb2_v, ib0_v, ib1_v, ib2_v, shared_den):
  c = lax.axis_index("c")
  s = lax.axis_index("s")
  wid = s * _NC + c
  ept = e // _NW
  ch = 2000
  gp = ((ch + 127) // 128) * 128
  base = wid * ept
  iota = lax.broadcasted_iota(I32, (16,), 0)
  zeros16 = jnp.zeros((16,), F32)
  zi16 = jnp.zeros((16,), I32)
  ex_bufs = (ex0_v, ex1_v, ex2_v)
  eb_bufs = (eb0_v, eb1_v, eb2_v)
  ib_bufs = (ib0_v, ib1_v, ib2_v)

  def zbuf(i, _):
    ex0_v[pl.ds(i * 16, 16)] = zeros16
    return 0
  lax.fori_loop(0, gp // 16, zbuf, 0)
  stripe = (3 * _NP) // _NS
  pltpu.sync_copy(ex0_v.at[pl.ds(0, stripe)],
                  shared_den.at[pl.ds(s * stripe, stripe)])
  plsc.subcore_barrier()

  for k in range((gp - ch) // 16):
    src_v[pl.ds(ch + k * 16, 16)] = zi16
    dst_v[pl.ds(ch + k * 16, 16)] = zi16

  pltpu.sync_copy(alpha_hbm, alpha_v)
  ngroups = (ch + 127) // 128

  for chunk in range(ept // ch):
    cbase = base + chunk * ch
    pltpu.sync_copy(src_hbm.at[pl.ds(cbase, ch)], src_v.at[pl.ds(0, ch)])
    pltpu.sync_copy(dst_hbm.at[pl.ds(cbase, ch)], dst_v.at[pl.ds(0, ch)])

    def group(g, _):
      off = g * 128
      for u in range(8):
        pos = off + u * 16
        d16 = dst_v[pl.ds(pos, 16)]
        s16 = src_v[pl.ds(pos, 16)]
        valid = (pos + iota) < ch
        for h in range(3):
          ga = plsc.load_gather(alpha_v, [s16 * 8 + h])
          gb = plsc.load_gather(alpha_v, [d16 * 8 + (3 + h)])
          a = ga + gb
          a = jnp.where(a >= 0.0, a, a * NEG_SLOPE)
          ex = jnp.where(valid, jnp.exp(a), 0.0)
          ex_bufs[h][pl.ds(pos, 16)] = ex
          eb_bufs[h][pl.ds(u * 16, 16)] = ex
          ib_bufs[h][pl.ds(u * 16, 16)] = h * _NP + d16
      for h in range(3):
        pltpu.sync_copy(eb_bufs[h], shared_den.at[ib_bufs[h]], add=True)
      return 0

    lax.fori_loop(0, ngroups, group, 0)

    for h in range(3):
      pltpu.sync_copy(ex_bufs[h].at[pl.ds(0, ch)],
                      ex_hbm.at[pl.ds(h * e + cbase, ch)])

  plsc.subcore_barrier()
  pltpu.sync_copy(shared_den.at[pl.ds(s * stripe, stripe)],
                  denom_hbm.at[c, pl.ds(s * stripe, stripe)])


def _sc_b(src, dst, alpha_flat, n, e):
  gp = 2048
  body = functools.partial(_sc_b_body, n, e)
  return pl.kernel(
      body,
      out_type=[
          jax.ShapeDtypeStruct((3 * e,), F32),
          jax.ShapeDtypeStruct((_NC, 3 * _NP), F32),
      ],
      mesh=_mesh(),
      compiler_params=pltpu.CompilerParams(needs_layout_passes=False),
      scratch_types=[
          pltpu.VMEM((8 * n,), F32),
          pltpu.VMEM((gp,), I32),
          pltpu.VMEM((gp,), I32),
          pltpu.VMEM((gp,), F32),
          pltpu.VMEM((gp,), F32),
          pltpu.VMEM((gp,), F32),
          pltpu.VMEM((128,), F32),
          pltpu.VMEM((128,), F32),
          pltpu.VMEM((128,), F32),
          pltpu.VMEM((128,), I32),
          pltpu.VMEM((128,), I32),
          pltpu.VMEM((128,), I32),
          pltpu.VMEM_SHARED((3 * _NP,), F32),
      ],
  )(src, dst, alpha_flat)


def _sc_c_body(n, e, task, src_hbm, dst_hbm, ex_hbm, den_hbm, wv_hbm, res_hbm,
               den_v, src_v, dst_v, exh_v, gidx_v, sidx_v, sc_v, rows_v, acc):
  c = lax.axis_index("c")
  s = lax.axis_index("s")
  hn = n // 2
  nacc = acc.shape[0]
  iota = lax.broadcasted_iota(I32, (16,), 0)
  zeros16 = jnp.zeros((16,), F32)
  head = jnp.asarray(c * 2 if task == 0 else 1, I32)

  def zrow(i, _):
    for q in range(8):
      rows_v[i, pl.ds(q * 16, 16)] = zeros16
    return 0
  lax.fori_loop(0, 128, zrow, 0)

  stripe = nacc // _NS
  for k in range(stripe // 128):
    pltpu.sync_copy(rows_v, acc.at[pl.ds(s * stripe + k * 128, 128)])
  rem = stripe % 128
  if rem:
    pltpu.sync_copy(rows_v.at[pl.ds(0, rem)],
                    acc.at[pl.ds(s * stripe + stripe - rem, rem)])

  pltpu.sync_copy(den_hbm.at[pl.ds(head * _NP, _NP)], den_v)
  plsc.subcore_barrier()

  ept = e // _NS
  gbase = s * ept

  chunks = []
  off = 0
  while off + 4992 <= ept:
    chunks.append((off, 4992))
    off += 4992
  if off < ept:
    chunks.append((off, ept - off))

  for (coff, clen) in chunks:
    pltpu.sync_copy(src_hbm.at[pl.ds(gbase + coff, clen)],
                    src_v.at[pl.ds(0, clen)])
    pltpu.sync_copy(dst_hbm.at[pl.ds(gbase + coff, clen)],
                    dst_v.at[pl.ds(0, clen)])
    pltpu.sync_copy(ex_hbm.at[pl.ds(head * e + gbase + coff, clen)],
                    exh_v.at[pl.ds(0, clen)])
    ngroups = (clen + 127) // 128

    def group(g, _, clen=clen):
      off2 = g * 128
      for u in range(8):
        pos = off2 + u * 16
        d16 = dst_v[pl.ds(pos, 16)]
        s16 = src_v[pl.ds(pos, 16)]
        ex16 = exh_v[pl.ds(pos, 16)]
        den16 = plsc.load_gather(den_v, [d16])
        sc = ex16 / den16
        valid = (pos + iota) < clen
        if task == 0:
          row = d16
        else:
          in_hi = d16 >= hn
          row = jnp.where(in_hi, d16 - hn, d16)
          own = jnp.where(c == 0, jnp.logical_not(in_hi), in_hi)
          valid = jnp.logical_and(valid, own)
        sc = jnp.where(valid, sc, 0.0)
        sc_v[pl.ds(u * 16, 16)] = sc
        sidx_v[pl.ds(u * 16, 16)] = row
        gidx_v[pl.ds(u * 16, 16)] = head * n + s16
      pltpu.sync_copy(wv_hbm.at[gidx_v], rows_v)

      def scale(u, _):
        sv = sc_v[pl.ds(u * 16, 16)]
        for j in range(16):
          b = _splat(sv, j)
          r = u * 16 + j
          for q in range(8):
            rows_v[r, pl.ds(q * 16, 16)] = (
                rows_v[r, pl.ds(q * 16, 16)] * b)
        return 0
      lax.fori_loop(0, 8, scale, 0)

      pltpu.sync_copy(rows_v, acc.at[sidx_v], add=True)
      return 0

    lax.fori_loop(0, ngroups, group, 0)

  plsc.subcore_barrier()

  span = n if task == 0 else hn
  obase = c * span
  for k in range(5):
    start = s * 640 + k * 128

    @pl.when(start + 128 <= span)
    def _(start=start):
      pltpu.sync_copy(acc.at[pl.ds(start, 128)],
                      res_hbm.at[pl.ds(obase + start, 128)])

  tail = span - (span // 128) * 128
  if tail:
    @pl.when(s == _NS - 1)
    def _():
      pltpu.sync_copy(acc.at[pl.ds(span - tail, tail)],
                      res_hbm.at[pl.ds(obase + span - tail, tail)])


def _sc_c(src, dst, ex_flat, den_flat, wv_flat, n, e, task):
  nacc = 10240 if task == 0 else 5120
  nout = 2 * n if task == 0 else n
  body = functools.partial(_sc_c_body, n, e, task)
  return pl.kernel(
      body,
      out_type=jax.ShapeDtypeStruct((nout, 128), F32),
      mesh=_mesh(),
      compiler_params=pltpu.CompilerParams(needs_layout_passes=False),
      scratch_types=[
          pltpu.VMEM((_NP,), F32),
          pltpu.VMEM((4992,), I32),
          pltpu.VMEM((4992,), I32),
          pltpu.VMEM((4992,), F32),
          pltpu.VMEM((128,), I32),
          pltpu.VMEM((128,), I32),
          pltpu.VMEM((128,), F32),
          pltpu.VMEM((128, 128), F32),
          pltpu.VMEM_SHARED((nacc, 128), F32),
      ],
  )(src, dst, ex_flat, den_flat, wv_flat)


def kernel(h, edge_index, W_node, b_node, w_att, w_scale, bias, ln_gamma,
           ln_beta):
  n, f = h.shape
  heads = w_att.shape[1]
  e = edge_index.shape[1]
  assert f == 128 and heads == 3 and n % 2000 == 0 and e % (16 * _NW) == 0

  src = edge_index[0].astype(I32)
  dst = edge_index[1].astype(I32)

  a_u = w_att[0, :, :f]
  a_v = w_att[0, :, f:]
  a_cat = jnp.zeros((heads * f, 8), F32)
  for hh in range(heads):
    a_cat = a_cat.at[hh * f:(hh + 1) * f, hh].set(a_u[hh])
    a_cat = a_cat.at[hh * f:(hh + 1) * f, 3 + hh].set(a_v[hh])

  blk = 1000
  wv3, alpha_t = _tc_a(h, W_node, b_node.reshape(1, -1), a_cat, n, blk)
  ex_flat, denom_p = _sc_b(src, dst, alpha_t.reshape(-1), n, e)
  den_flat = _tc_b2(denom_p).reshape(-1)
  wv_flat = wv3.reshape(3 * n, 128)
  out1 = _sc_c(src, dst, ex_flat, den_flat, wv_flat, n, e, 0)
  out2 = _sc_c(src, dst, ex_flat, den_flat, wv_flat, n, e, 1)
  out = _tc_d(out1, out2, h, w_scale,
              bias.reshape(1, -1), ln_gamma.reshape(1, -1),
              ln_beta.reshape(1, -1), n, blk)
  return out


# speedup vs baseline: 63.4004x; 1.7527x over previous
import functools

import jax
import jax.numpy as jnp
from jax import lax
from jax.experimental import pallas as pl
from jax.experimental.pallas import tpu as pltpu
from jax.experimental.pallas import tpu_sc as plsc

F32 = jnp.float32
I32 = jnp.int32
NEG_SLOPE = 0.2

_NC, _NS = 2, 16
_NW = _NC * _NS
_NP = 10240


@functools.lru_cache(maxsize=None)
def _mesh():
  return plsc.VectorSubcoreMesh(core_axis_name="c", subcore_axis_name="s",
                                num_cores=_NC, num_subcores=_NS)


def _splat(vec, j):
  idx = jnp.full((16,), j, I32)
  return lax.gather(
      vec, idx[:, None],
      lax.GatherDimensionNumbers(offset_dims=(), collapsed_slice_dims=(0,),
                                 start_index_map=(0,)),
      slice_sizes=(1,), mode=lax.GatherScatterMode.PROMISE_IN_BOUNDS)


def _tc_a_body(h_ref, w_ref, b_ref, a_ref, wv_ref, al_ref):
  hb = h_ref[...]
  wv = jnp.dot(hb, w_ref[...], preferred_element_type=F32) + b_ref[...]
  al = jnp.dot(wv, a_ref[...], preferred_element_type=F32)
  blk = hb.shape[0]
  wv_ref[...] = wv.reshape(blk, 3, 128).transpose(1, 0, 2)
  al_ref[...] = al


def _tc_a(h, w_node, b2, a_cat, n, blk):
  grid = n // blk
  return pl.pallas_call(
      _tc_a_body,
      grid=(grid,),
      in_specs=[
          pl.BlockSpec((blk, 128), lambda i: (i, 0)),
          pl.BlockSpec((128, 384), lambda i: (0, 0)),
          pl.BlockSpec((1, 384), lambda i: (0, 0)),
          pl.BlockSpec((384, 8), lambda i: (0, 0)),
      ],
      out_specs=[
          pl.BlockSpec((3, blk, 128), lambda i: (0, i, 0)),
          pl.BlockSpec((blk, 8), lambda i: (i, 0)),
      ],
      out_shape=[
          jax.ShapeDtypeStruct((3, n, 128), F32),
          jax.ShapeDtypeStruct((n, 8), F32),
      ],
  )(h, w_node, b2, a_cat)


def _tc_b2_body(dp_ref, out_ref):
  out_ref[...] = jnp.sum(dp_ref[...], axis=0, keepdims=True)


def _tc_b2(denom_p):
  blk = 1024
  grid = 3 * _NP // blk
  return pl.pallas_call(
      _tc_b2_body,
      grid=(grid,),
      in_specs=[pl.BlockSpec((_NC, blk), lambda i: (0, i))],
      out_specs=pl.BlockSpec((1, blk), lambda i: (0, i)),
      out_shape=jax.ShapeDtypeStruct((1, 3 * _NP), F32),
  )(denom_p)


def _tc_d_body(h0_ref, h1a_ref, h1b_ref, h2_ref, h_ref, ws_ref, b_ref,
               g_ref, be_ref, out_ref):
  ws = ws_ref[...]
  acc = jnp.dot(h0_ref[...], ws[0:128], preferred_element_type=F32)
  acc += jnp.dot(h1a_ref[...] + h1b_ref[...], ws[128:256],
                 preferred_element_type=F32)
  acc += jnp.dot(h2_ref[...], ws[256:384], preferred_element_type=F32)
  y = acc + b_ref[...] + h_ref[...]
  mu = jnp.mean(y, axis=-1, keepdims=True)
  d = y - mu
  var = jnp.mean(d * d, axis=-1, keepdims=True)
  ln = d * jax.lax.rsqrt(var + 1e-5) * g_ref[...] + be_ref[...]
  out_ref[...] = jnp.maximum(ln, 0.0)


def _tc_d(out1, out2, h, w_scale, b2, g2, be2, n, blk):
  grid = n // blk
  nb = n // blk
  return pl.pallas_call(
      _tc_d_body,
      grid=(grid,),
      in_specs=[
          pl.BlockSpec((blk, 128), lambda i: (i, 0)),
          pl.BlockSpec((blk, 128), lambda i: (i, 0)),
          pl.BlockSpec((blk, 128), lambda i: (i + nb, 0)),
          pl.BlockSpec((blk, 128), lambda i: (i + nb, 0)),
          pl.BlockSpec((blk, 128), lambda i: (i, 0)),
          pl.BlockSpec((384, 128), lambda i: (0, 0)),
          pl.BlockSpec((1, 128), lambda i: (0, 0)),
          pl.BlockSpec((1, 128), lambda i: (0, 0)),
          pl.BlockSpec((1, 128), lambda i: (0, 0)),
      ],
      out_specs=pl.BlockSpec((blk, 128), lambda i: (i, 0)),
      out_shape=jax.ShapeDtypeStruct((n, 128), F32),
  )(out1, out2, out2, out1, h, w_scale, b2, g2, be2)


def _sc_b_body(n, e, src_hbm, dst_hbm, alpha_hbm, ex_hbm, denom_hbm,
               alpha_v, src_v, dst_v, ex0_v, ex1_v, ex2_v,
               eb0_v, eb1_v, eb2_v, ib0_v, ib1_v, ib2_v, shared_den):
  c = lax.axis_index("c")
  s = lax.axis_index("s")
  wid = s * _NC + c
  ept = e // _NW
  ch = 2000
  gp = ((ch + 127) // 128) * 128
  base = wid * ept
  iota = lax.broadcasted_iota(I32, (16,), 0)
  zeros16 = jnp.zeros((16,), F32)
  zi16 = jnp.zeros((16,), I32)
  ex_bufs = (ex0_v, ex1_v, ex2_v)
  eb_bufs = (eb0_v, eb1_v, eb2_v)
  ib_bufs = (ib0_v, ib1_v, ib2_v)

  def zbuf(i, _):
    ex0_v[pl.ds(i * 16, 16)] = zeros16
    return 0
  lax.fori_loop(0, gp // 16, zbuf, 0)
  stripe = (3 * _NP) // _NS
  pltpu.sync_copy(ex0_v.at[pl.ds(0, stripe)],
                  shared_den.at[pl.ds(s * stripe, stripe)])
  plsc.subcore_barrier()

  for k in range((gp - ch) // 16):
    src_v[pl.ds(ch + k * 16, 16)] = zi16
    dst_v[pl.ds(ch + k * 16, 16)] = zi16

  pltpu.sync_copy(alpha_hbm, alpha_v)
  ngroups = (ch + 127) // 128

  for chunk in range(ept // ch):
    cbase = base + chunk * ch
    pltpu.sync_copy(src_hbm.at[pl.ds(cbase, ch)], src_v.at[pl.ds(0, ch)])
    pltpu.sync_copy(dst_hbm.at[pl.ds(cbase, ch)], dst_v.at[pl.ds(0, ch)])

    def group(g, _):
      off = g * 128
      for u in range(8):
        pos = off + u * 16
        d16 = dst_v[pl.ds(pos, 16)]
        s16 = src_v[pl.ds(pos, 16)]
        valid = (pos + iota) < ch
        for h in range(3):
          ga = plsc.load_gather(alpha_v, [s16 * 8 + h])
          gb = plsc.load_gather(alpha_v, [d16 * 8 + (3 + h)])
          a = ga + gb
          a = jnp.where(a >= 0.0, a, a * NEG_SLOPE)
          ex = jnp.where(valid, jnp.exp(a), 0.0)
          ex_bufs[h][pl.ds(pos, 16)] = ex
          eb_bufs[h][pl.ds(u * 16, 16)] = ex
          ib_bufs[h][pl.ds(u * 16, 16)] = h * _NP + d16
      for h in range(3):
        pltpu.sync_copy(eb_bufs[h], shared_den.at[ib_bufs[h]], add=True)
      return 0

    lax.fori_loop(0, ngroups, group, 0)

    for h in range(3):
      pltpu.sync_copy(ex_bufs[h].at[pl.ds(0, ch)],
                      ex_hbm.at[pl.ds(h * e + cbase, ch)])

  plsc.subcore_barrier()
  pltpu.sync_copy(shared_den.at[pl.ds(s * stripe, stripe)],
                  denom_hbm.at[c, pl.ds(s * stripe, stripe)])


def _sc_b(src, dst, alpha_flat, n, e):
  gp = 2048
  body = functools.partial(_sc_b_body, n, e)
  return pl.kernel(
      body,
      out_type=[
          jax.ShapeDtypeStruct((3 * e,), F32),
          jax.ShapeDtypeStruct((_NC, 3 * _NP), F32),
      ],
      mesh=_mesh(),
      compiler_params=pltpu.CompilerParams(needs_layout_passes=False),
      scratch_types=[
          pltpu.VMEM((8 * n,), F32),
          pltpu.VMEM((gp,), I32),
          pltpu.VMEM((gp,), I32),
          pltpu.VMEM((gp,), F32),
          pltpu.VMEM((gp,), F32),
          pltpu.VMEM((gp,), F32),
          pltpu.VMEM((128,), F32),
          pltpu.VMEM((128,), F32),
          pltpu.VMEM((128,), F32),
          pltpu.VMEM((128,), I32),
          pltpu.VMEM((128,), I32),
          pltpu.VMEM((128,), I32),
          pltpu.VMEM_SHARED((3 * _NP,), F32),
      ],
  )(src, dst, alpha_flat)


def _sc_c_body(n, e, src_hbm, dst_hbm, ex_hbm, den_hbm, wv_hbm,
               out1_hbm, out2_hbm,
               den_v, src_v, dst_v, exh_v,
               gidx_a, sidx_a, sc_a, rows_a, gidx_b, sidx_b, sc_b, rows_b,
               sem_a, sem_b, acc):
  c = lax.axis_index("c")
  s = lax.axis_index("s")
  iota = lax.broadcasted_iota(I32, (16,), 0)
  zeros16 = jnp.zeros((16,), F32)
  zi16 = jnp.zeros((16,), I32)

  def zero_acc_stripe():
    for k in range(8):
      st = s * 640 + k * 80

      @pl.when(st + 80 <= n)
      def _(st=st):
        pltpu.sync_copy(rows_a.at[pl.ds(0, 80)], acc.at[pl.ds(st, 80)])

  def zrow(i, _):
    for q in range(8):
      rows_a[i, pl.ds(q * 16, 16)] = zeros16
    return 0
  lax.fori_loop(0, 96, zrow, 0)
  zero_acc_stripe()

  def build(off2, clen, gidx_v, sidx_v, sc_v, head):
    for u in range(6):
      pos = off2 + u * 16
      d16 = dst_v[pl.ds(pos, 16)]
      s16 = src_v[pl.ds(pos, 16)]
      ex16 = exh_v[pl.ds(pos, 16)]
      den16 = plsc.load_gather(den_v, [d16])
      sc = ex16 / den16
      valid = (pos + iota) < clen
      sc = jnp.where(valid, sc, 0.0)
      sc_v[pl.ds(u * 16, 16)] = sc
      sidx_v[pl.ds(u * 16, 16)] = d16
      gidx_v[pl.ds(u * 16, 16)] = head * n + s16

  def scale_and_scatter(gidx_v, sidx_v, sc_v, rows_v, sem):
    pltpu.make_async_copy(wv_hbm.at[gidx_v], rows_v, sem).wait()

    def scale(u, _):
      sv = sc_v[pl.ds(u * 16, 16)]
      for j in range(16):
        b = _splat(sv, j)
        r = u * 16 + j
        for q in range(8):
          rows_v[r, pl.ds(q * 16, 16)] = rows_v[r, pl.ds(q * 16, 16)] * b
      return 0
    lax.fori_loop(0, 6, scale, 0)
    pltpu.sync_copy(rows_v, acc.at[sidx_v], add=True)

  def start(gidx_v, rows_v, sem):
    pltpu.async_copy(wv_hbm.at[gidx_v], rows_v, sem)

  for task in range(2):
    head = jnp.asarray(c * 2 if task == 0 else 1, I32)
    if task == 0:
      ept = e // _NS
      gbase = s * ept
      res_hbm = out1_hbm
    else:
      ept = e // _NW
      gbase = (s * _NC + c) * ept
      res_hbm = out2_hbm

    pltpu.sync_copy(den_hbm.at[pl.ds(head * _NP, n)], den_v)
    plsc.subcore_barrier()

    chunks = []
    off = 0
    while off + 2496 <= ept:
      chunks.append((off, 2496))
      off += 2496
    if off < ept:
      chunks.append((off, ept - off))

    for (coff, clen) in chunks:
      ccap = ((clen + 95) // 96) * 96
      pltpu.sync_copy(src_hbm.at[pl.ds(gbase + coff, clen)],
                      src_v.at[pl.ds(0, clen)])
      pltpu.sync_copy(dst_hbm.at[pl.ds(gbase + coff, clen)],
                      dst_v.at[pl.ds(0, clen)])
      pltpu.sync_copy(ex_hbm.at[pl.ds(head * e + gbase + coff, clen)],
                      exh_v.at[pl.ds(0, clen)])
      ngroups = ccap // 96
      if ngroups == 1:
        build(0, clen, gidx_a, sidx_a, sc_a, head)
        start(gidx_a, rows_a, sem_a)
        scale_and_scatter(gidx_a, sidx_a, sc_a, rows_a, sem_a)
        continue
      gpad = ngroups + (ngroups % 2)
      npairs = gpad // 2

      build(0, clen, gidx_a, sidx_a, sc_a, head)
      start(gidx_a, rows_a, sem_a)

      def pair(p, _, clen=clen, npairs=npairs, head=head):
        build((2 * p + 1) * 96, clen, gidx_b, sidx_b, sc_b, head)
        start(gidx_b, rows_b, sem_b)
        scale_and_scatter(gidx_a, sidx_a, sc_a, rows_a, sem_a)

        @pl.when(p + 1 < npairs)
        def _():
          build((2 * p + 2) * 96, clen, gidx_a, sidx_a, sc_a, head)
          start(gidx_a, rows_a, sem_a)

        scale_and_scatter(gidx_b, sidx_b, sc_b, rows_b, sem_b)
        return 0

      lax.fori_loop(0, npairs, pair, 0)

    plsc.subcore_barrier()

    obase = c * n
    for k in range(8):
      st = s * 640 + k * 80

      @pl.when(st + 80 <= n)
      def _(st=st, res_hbm=res_hbm):
        pltpu.sync_copy(acc.at[pl.ds(st, 80)],
                        res_hbm.at[pl.ds(obase + st, 80)])

    if task == 0:
      def zrow2(i, _):
        for q in range(8):
          rows_a[i, pl.ds(q * 16, 16)] = zeros16
        return 0
      lax.fori_loop(0, 96, zrow2, 0)
      zero_acc_stripe()
      plsc.subcore_barrier()


def _sc_c(src, dst, ex_flat, den_flat, wv_flat, n, e):
  body = functools.partial(_sc_c_body, n, e)
  return pl.kernel(
      body,
      out_type=[
          jax.ShapeDtypeStruct((2 * n, 128), F32),
          jax.ShapeDtypeStruct((2 * n, 128), F32),
      ],
      mesh=_mesh(),
      compiler_params=pltpu.CompilerParams(needs_layout_passes=False),
      scratch_types=[
          pltpu.VMEM((n,), F32),
          pltpu.VMEM((2496,), I32),
          pltpu.VMEM((2496,), I32),
          pltpu.VMEM((2496,), F32),
          pltpu.VMEM((96,), I32),
          pltpu.VMEM((96,), I32),
          pltpu.VMEM((96,), F32),
          pltpu.VMEM((96, 128), F32),
          pltpu.VMEM((96,), I32),
          pltpu.VMEM((96,), I32),
          pltpu.VMEM((96,), F32),
          pltpu.VMEM((96, 128), F32),
          pltpu.SemaphoreType.DMA,
          pltpu.SemaphoreType.DMA,
          pltpu.VMEM_SHARED((n, 128), F32),
      ],
  )(src, dst, ex_flat, den_flat, wv_flat)


def kernel(h, edge_index, W_node, b_node, w_att, w_scale, bias, ln_gamma,
           ln_beta):
  n, f = h.shape
  heads = w_att.shape[1]
  e = edge_index.shape[1]
  assert f == 128 and heads == 3 and n % 2000 == 0 and e % (16 * _NW) == 0

  src = edge_index[0].astype(I32)
  dst = edge_index[1].astype(I32)

  a_u = w_att[0, :, :f]
  a_v = w_att[0, :, f:]
  a_cat = jnp.zeros((heads * f, 8), F32)
  for hh in range(heads):
    a_cat = a_cat.at[hh * f:(hh + 1) * f, hh].set(a_u[hh])
    a_cat = a_cat.at[hh * f:(hh + 1) * f, 3 + hh].set(a_v[hh])

  blk = 1000
  wv3, alpha_t = _tc_a(h, W_node, b_node.reshape(1, -1), a_cat, n, blk)
  ex_flat, denom_p = _sc_b(src, dst, alpha_t.reshape(-1), n, e)
  den_flat = _tc_b2(denom_p).reshape(-1)
  wv_flat = wv3.reshape(3 * n, 128)
  out1, out2 = _sc_c(src, dst, ex_flat, den_flat, wv_flat, n, e)
  out = _tc_d(out1, out2, h, w_scale,
              bias.reshape(1, -1), ln_gamma.reshape(1, -1),
              ln_beta.reshape(1, -1), n, blk)
  return out
